# baseline (speedup 1.0000x reference)
import functools

import jax
import jax.numpy as jnp
from jax import lax
from jax.experimental import pallas as pl
from jax.experimental.pallas import tpu as pltpu
from jax.experimental.pallas import tpu_sc as plsc

B = 4096
NCAT = 26
VCAT = 100000
VSEQ = 1000000
L = 50
D = 32
NCONT = 13
HID = 128

NW = 32
BPW = B // NW
SEQ_PW = BPW * L
CHUNK = 128
NSEQ_CHUNKS = SEQ_PW // CHUNK

_sc_mesh = plsc.VectorSubcoreMesh(core_axis_name="c", subcore_axis_name="s")


@functools.partial(
    pl.kernel,
    mesh=_sc_mesh,
    compiler_params=pltpu.CompilerParams(use_tc_tiling_on_sc=False,
                                         needs_layout_passes=False),
    out_type=[
        jax.ShapeDtypeStruct((D, NCAT, B), jnp.float32),
        jax.ShapeDtypeStruct((D, B * L), jnp.float32),
    ],
    scratch_types=[
        pltpu.VMEM((NCAT, CHUNK), jnp.int32),
        pltpu.VMEM((NSEQ_CHUNKS, CHUNK), jnp.int32),
        pltpu.VMEM((D, CHUNK), jnp.float32),
        pltpu.SemaphoreType.DMA,
    ],
)
def _sc_gather(cat_idx_hbm, seq_idx_hbm, cat_tab_hbm, seq_tab_hbm,
               cat_out_hbm, seq_out_hbm,
               cidx_v, sidx_v, out_v, sem):
    sid = lax.axis_index("s")
    wid = sid * 2 + lax.axis_index("c")

    pltpu.sync_copy(cat_idx_hbm.at[wid], cidx_v)
    pltpu.sync_copy(seq_idx_hbm.at[wid], sidx_v)

    def cat_body(k, carry):
        copies = []
        for c in range(D):
            copies.append(
                pltpu.async_copy(cat_tab_hbm.at[k].at[c].at[cidx_v.at[k]],
                                 out_v.at[c], sem))
        for cp in copies:
            cp.wait()
        pltpu.sync_copy(out_v,
                        cat_out_hbm.at[:, k, pl.ds(wid * BPW, BPW)])
        return carry

    lax.fori_loop(0, NCAT, cat_body, 0)

    def seq_body(k, carry):
        copies = []
        for c in range(D):
            copies.append(
                pltpu.async_copy(seq_tab_hbm.at[c].at[sidx_v.at[k]],
                                 out_v.at[c], sem))
        for cp in copies:
            cp.wait()
        pltpu.sync_copy(
            out_v, seq_out_hbm.at[:, pl.ds(wid * SEQ_PW + k * CHUNK, CHUNK)])
        return carry

    lax.fori_loop(0, NSEQ_CHUNKS, seq_body, 0)


BLK = 512
CAT_F = NCAT * D


def _mlp_body(cat_ref, cont_ref, seq_ref, w1c_ref, w1x_ref, w1s_ref,
              b1_ref, w2_ref, b2_ref, out_ref):
    catm = cat_ref[...].reshape(D * NCAT, cat_ref.shape[2])
    hc = lax.dot_general(catm, w1c_ref[...], (((0,), (0,)), ((), ())),
                         preferred_element_type=jnp.float32)
    pooled = jnp.sum(seq_ref[...], axis=2) * (1.0 / L)
    hs = lax.dot_general(pooled, w1s_ref[...], (((0,), (0,)), ((), ())),
                         preferred_element_type=jnp.float32)
    hx = jnp.dot(cont_ref[...], w1x_ref[...],
                 preferred_element_type=jnp.float32)
    h = jnp.maximum(hc + hs + hx + b1_ref[...], 0.0)
    logits = jnp.sum(h * w2_ref[...], axis=1) + b2_ref[0, 0]
    out_ref[...] = logits[None, :]


def kernel(x_cat, x_cont, hist_seq, cat_tables, seq_table, W1, b1, W2, b2):
    cat_tab = cat_tables.transpose(0, 2, 1)
    seq_tab = seq_table.T

    xc = x_cat.T.reshape(NCAT, NW, BPW).transpose(1, 0, 2)
    hs = hist_seq.reshape(NW, NSEQ_CHUNKS, CHUNK)

    cat_t, seq_t = _sc_gather(xc, hs, cat_tab, seq_tab)

    seq3 = seq_t.reshape(D, B, L)

    w1c = W1[:CAT_F].reshape(NCAT, D, HID).transpose(1, 0, 2).reshape(CAT_F, HID)
    w1x = W1[CAT_F:CAT_F + NCONT]
    w1s = W1[CAT_F + NCONT:]

    out = pl.pallas_call(
        _mlp_body,
        grid=(B // BLK,),
        in_specs=[
            pl.BlockSpec((D, NCAT, BLK), lambda i: (0, 0, i)),
            pl.BlockSpec((BLK, NCONT), lambda i: (i, 0)),
            pl.BlockSpec((D, BLK, L), lambda i: (0, i, 0)),
            pl.BlockSpec((CAT_F, HID), lambda i: (0, 0)),
            pl.BlockSpec((NCONT, HID), lambda i: (0, 0)),
            pl.BlockSpec((D, HID), lambda i: (0, 0)),
            pl.BlockSpec((1, HID), lambda i: (0, 0)),
            pl.BlockSpec((1, HID), lambda i: (0, 0)),
            pl.BlockSpec((1, 1), lambda i: (0, 0)),
        ],
        out_specs=pl.BlockSpec((1, BLK), lambda i: (0, i)),
        out_shape=jax.ShapeDtypeStruct((1, B), jnp.float32),
    )(cat_t, x_cont, seq3,
      w1c, w1x, w1s, b1[None, :], W2.T, b2[None, :])

    return out[0]


# speedup vs baseline: 3.6926x; 3.6926x over previous
import functools

import jax
import jax.numpy as jnp
from jax import lax
from jax.experimental import pallas as pl
from jax.experimental.pallas import tpu as pltpu
from jax.experimental.pallas import tpu_sc as plsc

B = 4096
NCAT = 26
VCAT = 100000
VSEQ = 1000000
L = 50
D = 32
NCONT = 13
HID = 128

NW = 32
BPW = B // NW
SEQ_PW = BPW * L
CCHUNK = 128
SCHUNK = 128
NSEQ_CHUNKS = SEQ_PW // SCHUNK

_sc_mesh = plsc.VectorSubcoreMesh(core_axis_name="c", subcore_axis_name="s")
_sc_params = pltpu.CompilerParams(use_tc_tiling_on_sc=False,
                                  needs_layout_passes=False)


@functools.partial(
    pl.kernel,
    mesh=_sc_mesh,
    compiler_params=_sc_params,
    out_type=jax.ShapeDtypeStruct((D, NCAT, B), jnp.float32),
    scratch_types=[
        pltpu.VMEM((NCAT, CCHUNK), jnp.int32),
        pltpu.VMEM((D, CCHUNK), jnp.float32),
        pltpu.SemaphoreType.DMA,
    ],
)
def _sc_gather_cat(cat_idx_hbm, cat_tab_hbm, cat_out_hbm, cidx_v, out_v, sem):
    sid = lax.axis_index("s")
    wid = sid * 2 + lax.axis_index("c")
    pltpu.sync_copy(cat_idx_hbm.at[wid], cidx_v)

    def cat_body(k, carry):
        copies = []
        for c in range(D):
            copies.append(
                pltpu.async_copy(cat_tab_hbm.at[k].at[c].at[cidx_v.at[k]],
                                 out_v.at[c], sem))
        for cp in copies:
            cp.wait()
        pltpu.sync_copy(out_v,
                        cat_out_hbm.at[:, k, pl.ds(wid * BPW, BPW)])
        return carry

    lax.fori_loop(0, NCAT, cat_body, 0)


@functools.partial(
    pl.kernel,
    mesh=_sc_mesh,
    compiler_params=_sc_params,
    out_type=jax.ShapeDtypeStruct((B, D), jnp.float32),
    scratch_types=[
        pltpu.VMEM((NSEQ_CHUNKS, SCHUNK), jnp.int32),
        pltpu.VMEM((NSEQ_CHUNKS, SCHUNK), jnp.int32),
        pltpu.VMEM((SCHUNK, D), jnp.float32),
        pltpu.VMEM_SHARED((16 * BPW, D), jnp.float32),
        pltpu.VMEM((BPW, D), jnp.float32),
        pltpu.SemaphoreType.DMA,
        pltpu.SemaphoreType.DMA,
    ],
)
def _sc_gather_seq(seq_idx_hbm, pat_hbm, zeros_hbm, seq_tab_hbm, seq_out_hbm,
                   sidx_v, pat_v, rows_v, acc_shared, tmp_v, sem_g, sem_s):
    sid = lax.axis_index("s")
    wid = sid * 2 + lax.axis_index("c")

    pltpu.sync_copy(seq_idx_hbm.at[wid], sidx_v)
    pltpu.sync_copy(pat_hbm.at[sid], pat_v)
    pltpu.sync_copy(zeros_hbm, tmp_v)
    pltpu.sync_copy(tmp_v, acc_shared.at[pl.ds(sid * BPW, BPW)])

    def seq_body(k, carry):
        pltpu.async_copy(seq_tab_hbm.at[sidx_v.at[k]], rows_v, sem_g).wait()
        pltpu.async_copy(rows_v, acc_shared.at[pat_v.at[k]], sem_s,
                         add=True).wait()
        return carry

    lax.fori_loop(0, NSEQ_CHUNKS, seq_body, 0)

    pltpu.sync_copy(acc_shared.at[pl.ds(sid * BPW, BPW)], tmp_v)
    pltpu.sync_copy(tmp_v, seq_out_hbm.at[pl.ds(wid * BPW, BPW)])


BLK = 512
CAT_F = NCAT * D


def _mlp_body(cat_ref, cont_ref, seq_ref, w1c_ref, w1x_ref, w1s_ref,
              b1_ref, w2_ref, b2_ref, out_ref):
    catm = cat_ref[...].reshape(D * NCAT, cat_ref.shape[2])
    hc = lax.dot_general(catm, w1c_ref[...], (((0,), (0,)), ((), ())),
                         preferred_element_type=jnp.float32)
    hs = jnp.dot(seq_ref[...] * (1.0 / L), w1s_ref[...],
                 preferred_element_type=jnp.float32)
    hx = jnp.dot(cont_ref[...], w1x_ref[...],
                 preferred_element_type=jnp.float32)
    h = jnp.maximum(hc + hs + hx + b1_ref[...], 0.0)
    logits = jnp.sum(h * w2_ref[...], axis=1) + b2_ref[0, 0]
    out_ref[...] = logits[None, :]


def kernel(x_cat, x_cont, hist_seq, cat_tables, seq_table, W1, b1, W2, b2):
    cat_tab = cat_tables.transpose(0, 2, 1)

    xc = x_cat.T.reshape(NCAT, NW, BPW).transpose(1, 0, 2)
    hs = hist_seq.reshape(NW, NSEQ_CHUNKS, SCHUNK)
    base_pat = jnp.arange(SEQ_PW, dtype=jnp.int32) // L
    pat = (base_pat[None, :] + jnp.arange(16, dtype=jnp.int32)[:, None] * BPW
           ).reshape(16, NSEQ_CHUNKS, SCHUNK)
    zeros = jnp.zeros((BPW, D), jnp.float32)

    seq_sum = _sc_gather_seq(hs, pat, zeros, seq_table)
    cat_t = _sc_gather_cat(xc, cat_tab)

    w1c = W1[:CAT_F].reshape(NCAT, D, HID).transpose(1, 0, 2).reshape(CAT_F, HID)
    w1x = W1[CAT_F:CAT_F + NCONT]
    w1s = W1[CAT_F + NCONT:]

    out = pl.pallas_call(
        _mlp_body,
        grid=(B // BLK,),
        in_specs=[
            pl.BlockSpec((D, NCAT, BLK), lambda i: (0, 0, i)),
            pl.BlockSpec((BLK, NCONT), lambda i: (i, 0)),
            pl.BlockSpec((BLK, D), lambda i: (i, 0)),
            pl.BlockSpec((CAT_F, HID), lambda i: (0, 0)),
            pl.BlockSpec((NCONT, HID), lambda i: (0, 0)),
            pl.BlockSpec((D, HID), lambda i: (0, 0)),
            pl.BlockSpec((1, HID), lambda i: (0, 0)),
            pl.BlockSpec((1, HID), lambda i: (0, 0)),
            pl.BlockSpec((1, 1), lambda i: (0, 0)),
        ],
        out_specs=pl.BlockSpec((1, BLK), lambda i: (0, i)),
        out_shape=jax.ShapeDtypeStruct((1, B), jnp.float32),
    )(cat_t, x_cont, seq_sum,
      w1c, w1x, w1s, b1[None, :], W2.T, b2[None, :])

    return out[0]
